# baseline (speedup 1.0000x reference)
import functools
import math

import numpy as np
import jax
import jax.numpy as jnp
from jax import lax
from jax.experimental import pallas as pl
from jax.experimental.pallas import tpu as pltpu

B, L, N, D = 16, 192, 16, 128
E, K, DFF = 4, 2, 256
PATCHES = [8, 6, 4, 2]
F = L // 2 + 1
FPAD = 128
PPAD = 96
PLANE = 128
TOPF = 3

f32 = jnp.float32
i32 = jnp.int32


def _build_constants():
    T = np.zeros((L, L), np.float64)
    for k in (4, 8, 12):
        for l in range(L):
            for t in range(l - (k - 1) // 2, l + k // 2 + 1):
                j = min(max(t, 0), L - 1)
                T[l, j] += 1.0 / (3.0 * k)
    ls = np.arange(L)
    fs = np.arange(F)
    ang = 2.0 * np.pi * np.outer(fs, ls) / L
    C = np.cos(ang)
    S = -np.sin(ang)
    w = np.full(F, 2.0)
    w[0] = 1.0
    w[-1] = 1.0
    CI = (w[:, None] * np.cos(ang)) / L
    SI = -(w[:, None] * np.sin(ang)) / L
    TT = T.T
    CT = np.zeros((L, FPAD)); CT[:, :F] = C.T
    ST = np.zeros((L, FPAD)); ST[:, :F] = S.T
    CIT = np.zeros((FPAD, L)); CIT[:F, :] = CI
    SIT = np.zeros((FPAD, L)); SIT[:F, :] = SI
    A = np.zeros((E, PPAD, L), np.float64)
    R = np.zeros((E, L, PLANE), np.float64)
    KM = np.zeros((E, 1, PLANE), np.float64)
    for e, p in enumerate(PATCHES):
        P = L // p
        for i in range(P):
            A[e, i, i * p:(i + 1) * p] = 1.0 / p
        for l in range(L):
            R[e, l, l // p] = 1.0
        KM[e, 0, P:] = -1e30
    cvt = lambda a: np.asarray(a, np.float32)
    return (cvt(TT), cvt(CT), cvt(ST), cvt(CIT), cvt(SIT),
            cvt(A), cvt(R), cvt(KM))


_TT, _CT, _ST, _CIT, _SIT, _AP, _RP, _KM = _build_constants()


def _first_max_mask(a, ii, sentinel):
    m = jnp.max(a, axis=1, keepdims=True)
    cand = jnp.where(a == m, ii, sentinel)
    i1 = jnp.min(cand, axis=1, keepdims=True)
    return (ii == i1), m, i1


def _gate_kernel(x0t_ref, tt_ref, ct_ref, st_ref, cit_ref, sit_ref,
                 wtile_ref, bstart_ref, wgp_ref,
                 idx_ref, gate_ref, bal_ref):
    x0t = x0t_ref[...]
    trend = jnp.dot(x0t, tt_ref[...], preferred_element_type=f32)
    re = jnp.dot(x0t, ct_ref[...], preferred_element_type=f32)
    im = jnp.dot(x0t, st_ref[...], preferred_element_type=f32)
    amp = jnp.sqrt(re * re + im * im)
    ii = lax.broadcasted_iota(i32, (B * N, FPAD), 1)
    a = jnp.where(ii < F, amp, -1.0)
    mask = jnp.zeros((B * N, FPAD), f32)
    for _ in range(TOPF):
        fsel, _, _ = _first_max_mask(a, ii, FPAD)
        mask = jnp.where(fsel, 1.0, mask)
        a = jnp.where(fsel, -2.0, a)
    season = (jnp.dot(re * mask, cit_ref[...], preferred_element_type=f32)
              + jnp.dot(im * mask, sit_ref[...], preferred_element_type=f32))
    newx = x0t + trend + season

    rowb = lax.broadcasted_iota(i32, (B, B * N), 0)
    colb = lax.broadcasted_iota(i32, (B, B * N), 1) // N
    wt = jnp.broadcast_to(wtile_ref[0:1, :], (B, B * N))
    G = jnp.where(rowb == colb, wt, 0.0)
    gin = jnp.dot(G, newx, preferred_element_type=f32) + bstart_ref[0, 0]
    logits = jnp.dot(gin, wgp_ref[...], preferred_element_type=f32)

    jj = lax.broadcasted_iota(i32, (B, FPAD), 1)
    lg = jnp.where(jj < E, logits, -1e30)
    f1, m1, i1 = _first_max_mask(lg, jj, FPAD)
    a2 = jnp.where(f1, -1e30, lg)
    f2, m2, i2 = _first_max_mask(a2, jj, FPAD)
    e2v = jnp.exp(m2 - m1)
    den = 1.0 + e2v
    g1 = 1.0 / den
    g2 = e2v / den

    gates = jnp.where(f1, g1, 0.0) + jnp.where(f2, g2, 0.0)
    mc = (jj[0:1, :] < E).astype(f32)
    imp = jnp.sum(gates, axis=0, keepdims=True)
    load = jnp.sum(jnp.where(f1 | f2, 1.0, 0.0), axis=0, keepdims=True)

    def cv2(v):
        mean = jnp.sum(v * mc) / E
        var = jnp.sum(((v - mean) ** 2) * mc) / (E - 1)
        return var / (mean * mean + 1e-10)

    bal = 0.01 * (cv2(imp) + cv2(load))
    bal_ref[...] = jnp.full((8, FPAD), bal, f32)
    idx_ref[...] = jnp.where(jj == 0, jnp.broadcast_to(i1, (B, FPAD)),
                             jnp.where(jj == 1, jnp.broadcast_to(i2, (B, FPAD)), 0)
                             ).astype(i32)
    gate_ref[...] = jnp.where(jj == 0, jnp.broadcast_to(g1, (B, FPAD)),
                              jnp.where(jj == 1, jnp.broadcast_to(g2, (B, FPAD)), 0.0))


def _expert_kernel(e_ref,
                   xt_ref, g3_ref, wq_ref, wk_ref, wv_ref, wo_ref,
                   w1_ref, b1_ref, w2_ref, b2_ref,
                   a_ref, r_ref, km_ref, ewt_ref, endb_ref,
                   out_ref):
    s = pl.program_id(0)
    kidx = lax.rem(s, 2)
    gate = g3_ref[0, 0, 0]
    bf = jnp.bfloat16
    Ae = a_ref[0].astype(bf)
    Re = r_ref[0].astype(bf)
    km = km_ref[0]
    wq = wq_ref[0].astype(bf); wk = wk_ref[0].astype(bf)
    wv = wv_ref[0].astype(bf); wo = wo_ref[0].astype(bf)
    inv_sqrt_d = 1.0 / math.sqrt(float(D))

    x2 = xt_ref[0]
    x2b = x2.astype(bf)
    ax = jnp.dot(Ae, x2b, preferred_element_type=f32)
    xg = jnp.concatenate([ax[:, n * D:(n + 1) * D] for n in range(N)],
                         axis=0).astype(bf)
    q = jnp.dot(xg, wq, preferred_element_type=f32).astype(bf)
    kk = jnp.dot(xg, wk, preferred_element_type=f32).astype(bf)
    v = jnp.dot(xg, wv, preferred_element_type=f32).astype(bf)
    zpad = jnp.zeros((PLANE - PPAD, D), bf)
    sc_list = []
    for n in range(N):
        qn = q[n * PPAD:(n + 1) * PPAD]
        kn = jnp.concatenate([kk[n * PPAD:(n + 1) * PPAD], zpad], axis=0)
        sc_list.append(lax.dot_general(qn, kn, (((1,), (1,)), ((), ())),
                                       preferred_element_type=f32))
    sc = jnp.concatenate(sc_list, axis=0) * inv_sqrt_d + km
    m = jnp.max(sc, axis=1, keepdims=True)
    p = jnp.exp(sc - m)
    p = (p / jnp.sum(p, axis=1, keepdims=True)).astype(bf)
    o_list = []
    for n in range(N):
        vn = jnp.concatenate([v[n * PPAD:(n + 1) * PPAD], zpad], axis=0)
        o_list.append(jnp.dot(p[n * PPAD:(n + 1) * PPAD], vn,
                              preferred_element_type=f32))
    oa = jnp.concatenate(o_list, axis=0).astype(bf)
    o = jnp.dot(oa, wo, preferred_element_type=f32).astype(bf)
    zp2 = jnp.zeros((PLANE - PPAD, D), bf)
    h2_list = []
    for n in range(N):
        on = jnp.concatenate([o[n * PPAD:(n + 1) * PPAD], zp2], axis=0)
        obn = jnp.dot(Re, on, preferred_element_type=f32)
        h2_list.append(x2[:, n * D:(n + 1) * D] + obn)
    h2 = jnp.concatenate(h2_list, axis=0)
    u = jnp.maximum(jnp.dot(h2.astype(bf), w1_ref[0].astype(bf),
                            preferred_element_type=f32)
                    + b1_ref[0], 0.0)
    y = jnp.dot(u.astype(bf), w2_ref[0].astype(bf),
                preferred_element_type=f32) + b2_ref[0]
    tot = (h2 + y) * gate
    tot2 = jnp.concatenate([tot[n * L:(n + 1) * L] for n in range(N)],
                           axis=1)

    @pl.when(kidx == 0)
    def _():
        ewt = ewt_ref[...]
        res = jnp.dot(ewt, x2, preferred_element_type=f32)
        endb = jnp.broadcast_to(endb_ref[...][:, 0:1], (L, N * D))
        out_ref[0] = res + endb + tot2

    @pl.when(kidx == 1)
    def _():
        out_ref[0] = out_ref[0] + tot2


def _gate_call(x0t, wtile, bstart, wgp):
    return pl.pallas_call(
        _gate_kernel,
        out_shape=[
            jax.ShapeDtypeStruct((B, FPAD), i32),
            jax.ShapeDtypeStruct((B, FPAD), f32),
            jax.ShapeDtypeStruct((8, FPAD), f32),
        ],
    )(x0t, _TT, _CT, _ST, _CIT, _SIT, wtile, bstart, wgp)


def _expert_call(e_flat, xt, g3, Wq, Wk, Wv, Wo, W1, b1r, W2, b2r, ewt, endb):
    nsteps = B * K
    bspec = lambda shape, imap: pl.BlockSpec(shape, imap)
    by_e = lambda s, e_ref: (e_ref[s], 0, 0)
    grid_spec = pltpu.PrefetchScalarGridSpec(
        num_scalar_prefetch=1,
        grid=(nsteps,),
        in_specs=[
            bspec((1, L, N * D), lambda s, e_ref: (s // 2, 0, 0)),
            bspec((1, 1, FPAD), lambda s, e_ref: (s, 0, 0)),
            bspec((1, D, D), by_e),
            bspec((1, D, D), by_e),
            bspec((1, D, D), by_e),
            bspec((1, D, D), by_e),
            bspec((1, D, DFF), by_e),
            bspec((1, 1, DFF), by_e),
            bspec((1, DFF, D), by_e),
            bspec((1, 1, D), by_e),
            bspec((1, PPAD, L), by_e),
            bspec((1, L, PLANE), by_e),
            bspec((1, 1, PLANE), by_e),
            bspec((L, L), lambda s, e_ref: (0, 0)),
            bspec((L, D), lambda s, e_ref: (0, 0)),
        ],
        out_specs=pl.BlockSpec((1, L, N * D), lambda s, e_ref: (s // 2, 0, 0)),
    )
    return pl.pallas_call(
        _expert_kernel,
        grid_spec=grid_spec,
        out_shape=jax.ShapeDtypeStruct((B, L, N * D), f32),
    )(e_flat, xt, g3, Wq, Wk, Wv, Wo, W1, b1r, W2, b2r, _AP, _RP, _KM, ewt, endb)


def kernel(x, W_start, b_start, w_gate, Wq, Wk, Wv, Wo, W1, b1, W2, b2, end_W, end_b):
    x0t = x[:, :, :, 0].transpose(0, 2, 1).reshape(B * N, L)
    wtile = jnp.broadcast_to(jnp.tile(W_start[:, 0], B)[None, :], (8, B * N))
    bstart = jnp.broadcast_to(b_start.reshape(1, 1), (8, FPAD))
    wgp = jnp.pad(w_gate, ((0, 0), (0, FPAD - E)))
    idx, gate, bal = _gate_call(x0t, wtile, bstart, wgp)

    e_flat = idx[:, :K].reshape(-1)
    g3 = jnp.broadcast_to(gate[:, :K].reshape(B * K, 1, 1), (B * K, 1, FPAD))
    x2d = x.reshape(B, L, N * D)
    b1r = b1[:, None, :]
    b2r = b2[:, None, :]
    ewt = end_W.T
    endb = jnp.broadcast_to(end_b[:, None], (L, D))
    o2d = _expert_call(e_flat, x2d, g3, Wq, Wk, Wv, Wo, W1, b1r, W2, b2r, ewt, endb)
    out = o2d.reshape(B, L, N, D)
    return out, bal[0, 0]


# speedup vs baseline: 1.1519x; 1.1519x over previous
import functools
import math

import numpy as np
import jax
import jax.numpy as jnp
from jax import lax
from jax.experimental import pallas as pl
from jax.experimental.pallas import tpu as pltpu

B, L, N, D = 16, 192, 16, 128
E, K, DFF = 4, 2, 256
PATCHES = [8, 6, 4, 2]
F = L // 2 + 1
FPAD = 128
PPAD = 96
PLANE = 128
TOPF = 3

f32 = jnp.float32
i32 = jnp.int32


def _build_constants():
    T = np.zeros((L, L), np.float64)
    for k in (4, 8, 12):
        for l in range(L):
            for t in range(l - (k - 1) // 2, l + k // 2 + 1):
                j = min(max(t, 0), L - 1)
                T[l, j] += 1.0 / (3.0 * k)
    ls = np.arange(L)
    fs = np.arange(F)
    ang = 2.0 * np.pi * np.outer(fs, ls) / L
    C = np.cos(ang)
    S = -np.sin(ang)
    w = np.full(F, 2.0)
    w[0] = 1.0
    w[-1] = 1.0
    CI = (w[:, None] * np.cos(ang)) / L
    SI = -(w[:, None] * np.sin(ang)) / L
    TT = T.T
    CT = np.zeros((L, FPAD)); CT[:, :F] = C.T
    ST = np.zeros((L, FPAD)); ST[:, :F] = S.T
    CIT = np.zeros((FPAD, L)); CIT[:F, :] = CI
    SIT = np.zeros((FPAD, L)); SIT[:F, :] = SI
    A = np.zeros((E, PPAD, L), np.float64)
    R = np.zeros((E, L, PLANE), np.float64)
    KM = np.zeros((E, 1, PLANE), np.float64)
    for e, p in enumerate(PATCHES):
        P = L // p
        for i in range(P):
            A[e, i, i * p:(i + 1) * p] = 1.0 / p
        for l in range(L):
            R[e, l, l // p] = 1.0
        KM[e, 0, P:] = -1e30
    cvt = lambda a: np.asarray(a, np.float32)
    return (cvt(TT), cvt(CT), cvt(ST), cvt(CIT), cvt(SIT),
            cvt(A), cvt(R), cvt(KM))


_TT, _CT, _ST, _CIT, _SIT, _AP, _RP, _KM = _build_constants()


def _first_max_mask(a, ii, sentinel):
    m = jnp.max(a, axis=1, keepdims=True)
    cand = jnp.where(a == m, ii, sentinel)
    i1 = jnp.min(cand, axis=1, keepdims=True)
    return (ii == i1), m, i1


def _gate_kernel(x0t_ref, tt_ref, ct_ref, st_ref, cit_ref, sit_ref,
                 wtile_ref, bstart_ref, wgp_ref,
                 idx_ref, gate_ref, bal_ref):
    x0t = x0t_ref[...]
    trend = jnp.dot(x0t, tt_ref[...], preferred_element_type=f32)
    re = jnp.dot(x0t, ct_ref[...], preferred_element_type=f32)
    im = jnp.dot(x0t, st_ref[...], preferred_element_type=f32)
    amp = jnp.sqrt(re * re + im * im)
    ii = lax.broadcasted_iota(i32, (B * N, FPAD), 1)
    a = jnp.where(ii < F, amp, -1.0)
    mask = jnp.zeros((B * N, FPAD), f32)
    for _ in range(TOPF):
        fsel, _, _ = _first_max_mask(a, ii, FPAD)
        mask = jnp.where(fsel, 1.0, mask)
        a = jnp.where(fsel, -2.0, a)
    season = (jnp.dot(re * mask, cit_ref[...], preferred_element_type=f32)
              + jnp.dot(im * mask, sit_ref[...], preferred_element_type=f32))
    newx = x0t + trend + season

    rowb = lax.broadcasted_iota(i32, (B, B * N), 0)
    colb = lax.broadcasted_iota(i32, (B, B * N), 1) // N
    wt = jnp.broadcast_to(wtile_ref[0:1, :], (B, B * N))
    G = jnp.where(rowb == colb, wt, 0.0)
    gin = jnp.dot(G, newx, preferred_element_type=f32) + bstart_ref[0, 0]
    logits = jnp.dot(gin, wgp_ref[...], preferred_element_type=f32)

    jj = lax.broadcasted_iota(i32, (B, FPAD), 1)
    lg = jnp.where(jj < E, logits, -1e30)
    f1, m1, i1 = _first_max_mask(lg, jj, FPAD)
    a2 = jnp.where(f1, -1e30, lg)
    f2, m2, i2 = _first_max_mask(a2, jj, FPAD)
    e2v = jnp.exp(m2 - m1)
    den = 1.0 + e2v
    g1 = 1.0 / den
    g2 = e2v / den

    gates = jnp.where(f1, g1, 0.0) + jnp.where(f2, g2, 0.0)
    mc = (jj[0:1, :] < E).astype(f32)
    imp = jnp.sum(gates, axis=0, keepdims=True)
    load = jnp.sum(jnp.where(f1 | f2, 1.0, 0.0), axis=0, keepdims=True)

    def cv2(v):
        mean = jnp.sum(v * mc) / E
        var = jnp.sum(((v - mean) ** 2) * mc) / (E - 1)
        return var / (mean * mean + 1e-10)

    bal = 0.01 * (cv2(imp) + cv2(load))
    bal_ref[...] = jnp.full((8, FPAD), bal, f32)
    idx_ref[...] = jnp.where(jj == 0, jnp.broadcast_to(i1, (B, FPAD)),
                             jnp.where(jj == 1, jnp.broadcast_to(i2, (B, FPAD)), 0)
                             ).astype(i32)
    gate_ref[...] = jnp.where(jj == 0, jnp.broadcast_to(g1, (B, FPAD)),
                              jnp.where(jj == 1, jnp.broadcast_to(g2, (B, FPAD)), 0.0))


def _one_expert(xn_list, gate, wrefs):
    (wq_ref, wk_ref, wv_ref, wo_ref, w1_ref, b1_ref, w2_ref, b2_ref,
     a_ref, r_ref, km_ref) = wrefs
    bf = jnp.bfloat16
    Ae = a_ref[0].astype(bf)
    Re = r_ref[0].astype(bf)
    km = km_ref[0]
    wq = wq_ref[0].astype(bf); wk = wk_ref[0].astype(bf)
    wv = wv_ref[0].astype(bf); wo = wo_ref[0].astype(bf)
    inv_sqrt_d = 1.0 / math.sqrt(float(D))

    xg_list = []
    for n in range(N):
        xg_list.append(jnp.dot(Ae, xn_list[n].astype(bf),
                               preferred_element_type=f32))
    xg = jnp.concatenate(xg_list, axis=0).astype(bf)
    q = jnp.dot(xg, wq, preferred_element_type=f32).astype(bf)
    kk = jnp.dot(xg, wk, preferred_element_type=f32).astype(bf)
    v = jnp.dot(xg, wv, preferred_element_type=f32).astype(bf)
    zpad = jnp.zeros((PLANE - PPAD, D), bf)
    sc_list = []
    for n in range(N):
        qn = q[n * PPAD:(n + 1) * PPAD]
        kn = jnp.concatenate([kk[n * PPAD:(n + 1) * PPAD], zpad], axis=0)
        sc_list.append(lax.dot_general(qn, kn, (((1,), (1,)), ((), ())),
                                       preferred_element_type=f32))
    sc = jnp.concatenate(sc_list, axis=0) * inv_sqrt_d + km
    m = jnp.max(sc, axis=1, keepdims=True)
    p = jnp.exp(sc - m)
    p = (p / jnp.sum(p, axis=1, keepdims=True)).astype(bf)
    o_list = []
    for n in range(N):
        vn = jnp.concatenate([v[n * PPAD:(n + 1) * PPAD], zpad], axis=0)
        o_list.append(jnp.dot(p[n * PPAD:(n + 1) * PPAD], vn,
                              preferred_element_type=f32))
    oa = jnp.concatenate(o_list, axis=0).astype(bf)
    o = jnp.dot(oa, wo, preferred_element_type=f32).astype(bf)
    zp2 = jnp.zeros((PLANE - PPAD, D), bf)
    h2_list = []
    for n in range(N):
        on = jnp.concatenate([o[n * PPAD:(n + 1) * PPAD], zp2], axis=0)
        obn = jnp.dot(Re, on, preferred_element_type=f32)
        h2_list.append(xn_list[n] + obn)
    h2 = jnp.concatenate(h2_list, axis=0)
    u = jnp.maximum(jnp.dot(h2.astype(bf), w1_ref[0].astype(bf),
                            preferred_element_type=f32)
                    + b1_ref[0], 0.0)
    y = jnp.dot(u.astype(bf), w2_ref[0].astype(bf),
                preferred_element_type=f32) + b2_ref[0]
    return (h2 + y) * gate


def _expert_kernel(e_ref,
                   xt_ref, g3_ref,
                   wq_a, wk_a, wv_a, wo_a, w1_a, b1_a, w2_a, b2_a,
                   a_a, r_a, km_a,
                   wq_b, wk_b, wv_b, wo_b, w1_b, b1_b, w2_b, b2_b,
                   a_b, r_b, km_b,
                   ewt_ref, endb_ref,
                   out_ref):
    xn_list = [xt_ref[0, n] for n in range(N)]
    tot_a = _one_expert(xn_list, g3_ref[0, 0, 0],
                        (wq_a, wk_a, wv_a, wo_a, w1_a, b1_a, w2_a, b2_a,
                         a_a, r_a, km_a))
    tot_b = _one_expert(xn_list, g3_ref[0, 0, 1],
                        (wq_b, wk_b, wv_b, wo_b, w1_b, b1_b, w2_b, b2_b,
                         a_b, r_b, km_b))
    tot = tot_a + tot_b
    ewt = ewt_ref[...]
    endb = endb_ref[...]
    for n in range(N):
        resn = jnp.dot(ewt, xn_list[n], preferred_element_type=f32)
        out_ref[0, n] = resn + endb + tot[n * L:(n + 1) * L]


def _gate_call(x0t, wtile, bstart, wgp):
    return pl.pallas_call(
        _gate_kernel,
        out_shape=[
            jax.ShapeDtypeStruct((B, FPAD), i32),
            jax.ShapeDtypeStruct((B, FPAD), f32),
            jax.ShapeDtypeStruct((8, FPAD), f32),
        ],
    )(x0t, _TT, _CT, _ST, _CIT, _SIT, wtile, bstart, wgp)


def _expert_call(e_flat, xt, g3, Wq, Wk, Wv, Wo, W1, b1r, W2, b2r, ewt, endb):
    bspec = lambda shape, imap: pl.BlockSpec(shape, imap)
    by_ea = lambda s, e_ref: (e_ref[2 * s], 0, 0)
    by_eb = lambda s, e_ref: (e_ref[2 * s + 1], 0, 0)

    def wspecs(by_e):
        return [
            bspec((1, D, D), by_e),
            bspec((1, D, D), by_e),
            bspec((1, D, D), by_e),
            bspec((1, D, D), by_e),
            bspec((1, D, DFF), by_e),
            bspec((1, 1, DFF), by_e),
            bspec((1, DFF, D), by_e),
            bspec((1, 1, D), by_e),
            bspec((1, PPAD, L), by_e),
            bspec((1, L, PLANE), by_e),
            bspec((1, 1, PLANE), by_e),
        ]

    grid_spec = pltpu.PrefetchScalarGridSpec(
        num_scalar_prefetch=1,
        grid=(B,),
        in_specs=(
            [bspec((1, N, L, D), lambda s, e_ref: (s, 0, 0, 0)),
             bspec((1, 1, FPAD), lambda s, e_ref: (s, 0, 0))]
            + wspecs(by_ea) + wspecs(by_eb)
            + [bspec((L, L), lambda s, e_ref: (0, 0)),
               bspec((L, D), lambda s, e_ref: (0, 0))]
        ),
        out_specs=pl.BlockSpec((1, N, L, D), lambda s, e_ref: (s, 0, 0, 0)),
    )
    wargs = (Wq, Wk, Wv, Wo, W1, b1r, W2, b2r, _AP, _RP, _KM)
    return pl.pallas_call(
        _expert_kernel,
        grid_spec=grid_spec,
        out_shape=jax.ShapeDtypeStruct((B, N, L, D), f32),
    )(e_flat, xt, g3, *wargs, *wargs, ewt, endb)


def kernel(x, W_start, b_start, w_gate, Wq, Wk, Wv, Wo, W1, b1, W2, b2, end_W, end_b):
    x0t = x[:, :, :, 0].transpose(0, 2, 1).reshape(B * N, L)
    wtile = jnp.broadcast_to(jnp.tile(W_start[:, 0], B)[None, :], (8, B * N))
    bstart = jnp.broadcast_to(b_start.reshape(1, 1), (8, FPAD))
    wgp = jnp.pad(w_gate, ((0, 0), (0, FPAD - E)))
    idx, gate, bal = _gate_call(x0t, wtile, bstart, wgp)

    e_flat = idx[:, :K].reshape(-1)
    g3 = gate[:, None, :]
    xt = x.transpose(0, 2, 1, 3)
    b1r = b1[:, None, :]
    b2r = b2[:, None, :]
    ewt = end_W.T
    endb = jnp.broadcast_to(end_b[:, None], (L, D))
    ot = _expert_call(e_flat, xt, g3, Wq, Wk, Wv, Wo, W1, b1r, W2, b2r, ewt, endb)
    out = ot.transpose(0, 2, 1, 3)
    return out, bal[0, 0]


# speedup vs baseline: 1.4527x; 1.2611x over previous
import functools
import math

import numpy as np
import jax
import jax.numpy as jnp
from jax import lax
from jax.experimental import pallas as pl
from jax.experimental.pallas import tpu as pltpu

B, L, N, D = 16, 192, 16, 128
E, K, DFF = 4, 2, 256
PATCHES = [8, 6, 4, 2]
F = L // 2 + 1
FPAD = 128
PPAD = 96
PLANE = 128
TOPF = 3

f32 = jnp.float32
i32 = jnp.int32


def _build_constants():
    T = np.zeros((L, L), np.float64)
    for k in (4, 8, 12):
        for l in range(L):
            for t in range(l - (k - 1) // 2, l + k // 2 + 1):
                j = min(max(t, 0), L - 1)
                T[l, j] += 1.0 / (3.0 * k)
    ls = np.arange(L)
    fs = np.arange(F)
    ang = 2.0 * np.pi * np.outer(fs, ls) / L
    C = np.cos(ang)
    S = -np.sin(ang)
    w = np.full(F, 2.0)
    w[0] = 1.0
    w[-1] = 1.0
    CI = (w[:, None] * np.cos(ang)) / L
    SI = -(w[:, None] * np.sin(ang)) / L
    TT = T.T
    CT = np.zeros((L, FPAD)); CT[:, :F] = C.T
    ST = np.zeros((L, FPAD)); ST[:, :F] = S.T
    CIT = np.zeros((FPAD, L)); CIT[:F, :] = CI
    SIT = np.zeros((FPAD, L)); SIT[:F, :] = SI
    A = np.zeros((E, PPAD, L), np.float64)
    R = np.zeros((E, L, PLANE), np.float64)
    KM = np.zeros((E, 1, PLANE), np.float64)
    for e, p in enumerate(PATCHES):
        P = L // p
        for i in range(P):
            A[e, i, i * p:(i + 1) * p] = 1.0 / p
        for l in range(L):
            R[e, l, l // p] = 1.0
        KM[e, 0, P:] = -1e30
    cvt = lambda a: np.asarray(a, np.float32)
    return (cvt(TT), cvt(CT), cvt(ST), cvt(CIT), cvt(SIT),
            cvt(A), cvt(R), cvt(KM))


_TT, _CT, _ST, _CIT, _SIT, _AP, _RP, _KM = _build_constants()


def _first_max_mask(a, ii, sentinel):
    m = jnp.max(a, axis=1, keepdims=True)
    cand = jnp.where(a == m, ii, sentinel)
    i1 = jnp.min(cand, axis=1, keepdims=True)
    return (ii == i1), m, i1


def _gate_kernel(x0t_ref, tt_ref, ct_ref, st_ref, cit_ref, sit_ref,
                 wtile_ref, bstart_ref, wgp_ref,
                 idx_ref, gate_ref, bal_ref):
    x0t = x0t_ref[...]
    trend = jnp.dot(x0t, tt_ref[...], preferred_element_type=f32)
    re = jnp.dot(x0t, ct_ref[...], preferred_element_type=f32)
    im = jnp.dot(x0t, st_ref[...], preferred_element_type=f32)
    amp = jnp.sqrt(re * re + im * im)
    ii = lax.broadcasted_iota(i32, (B * N, FPAD), 1)
    a = jnp.where(ii < F, amp, -1.0)
    mask = jnp.zeros((B * N, FPAD), f32)
    for _ in range(TOPF):
        fsel, _, _ = _first_max_mask(a, ii, FPAD)
        mask = jnp.where(fsel, 1.0, mask)
        a = jnp.where(fsel, -2.0, a)
    season = (jnp.dot(re * mask, cit_ref[...], preferred_element_type=f32)
              + jnp.dot(im * mask, sit_ref[...], preferred_element_type=f32))
    newx = x0t + trend + season

    rowb = lax.broadcasted_iota(i32, (B, B * N), 0)
    colb = lax.broadcasted_iota(i32, (B, B * N), 1) // N
    wt = jnp.broadcast_to(wtile_ref[0:1, :], (B, B * N))
    G = jnp.where(rowb == colb, wt, 0.0)
    gin = jnp.dot(G, newx, preferred_element_type=f32) + bstart_ref[0, 0]
    logits = jnp.dot(gin, wgp_ref[...], preferred_element_type=f32)

    jj = lax.broadcasted_iota(i32, (B, FPAD), 1)
    lg = jnp.where(jj < E, logits, -1e30)
    f1, m1, i1 = _first_max_mask(lg, jj, FPAD)
    a2 = jnp.where(f1, -1e30, lg)
    f2, m2, i2 = _first_max_mask(a2, jj, FPAD)
    e2v = jnp.exp(m2 - m1)
    den = 1.0 + e2v
    g1 = 1.0 / den
    g2 = e2v / den

    gates = jnp.where(f1, g1, 0.0) + jnp.where(f2, g2, 0.0)
    mc = (jj[0:1, :] < E).astype(f32)
    imp = jnp.sum(gates, axis=0, keepdims=True)
    load = jnp.sum(jnp.where(f1 | f2, 1.0, 0.0), axis=0, keepdims=True)

    def cv2(v):
        mean = jnp.sum(v * mc) / E
        var = jnp.sum(((v - mean) ** 2) * mc) / (E - 1)
        return var / (mean * mean + 1e-10)

    bal = 0.01 * (cv2(imp) + cv2(load))
    bal_ref[...] = jnp.full((8, FPAD), bal, f32)
    idx_ref[...] = jnp.where(jj == 0, jnp.broadcast_to(i1, (B, FPAD)),
                             jnp.where(jj == 1, jnp.broadcast_to(i2, (B, FPAD)), 0)
                             ).astype(i32)
    gate_ref[...] = jnp.where(jj == 0, jnp.broadcast_to(g1, (B, FPAD)),
                              jnp.where(jj == 1, jnp.broadcast_to(g2, (B, FPAD)), 0.0))


def _one_expert(xn_list, gate, wrefs):
    (wq_ref, wk_ref, wv_ref, wo_ref, w1_ref, b1_ref, w2_ref, b2_ref,
     a_ref, r_ref, km_ref) = wrefs
    bf = jnp.bfloat16
    Ae = a_ref[0].astype(bf)
    Re = r_ref[0].astype(bf)
    km = km_ref[0]
    wq = wq_ref[0].astype(bf); wk = wk_ref[0].astype(bf)
    wv = wv_ref[0].astype(bf); wo = wo_ref[0].astype(bf)
    inv_sqrt_d = 1.0 / math.sqrt(float(D))

    xg_list = []
    for n in range(N):
        xg_list.append(jnp.dot(Ae, xn_list[n].astype(bf),
                               preferred_element_type=f32))
    xg = jnp.concatenate(xg_list, axis=0).astype(bf)
    q = jnp.dot(xg, wq, preferred_element_type=f32).astype(bf)
    kk = jnp.dot(xg, wk, preferred_element_type=f32).astype(bf)
    v = jnp.dot(xg, wv, preferred_element_type=f32).astype(bf)
    zpad = jnp.zeros((PLANE - PPAD, D), bf)
    sc_list = []
    for n in range(N):
        qn = q[n * PPAD:(n + 1) * PPAD]
        kn = jnp.concatenate([kk[n * PPAD:(n + 1) * PPAD], zpad], axis=0)
        sc_list.append(lax.dot_general(qn, kn, (((1,), (1,)), ((), ())),
                                       preferred_element_type=f32))
    sc = jnp.concatenate(sc_list, axis=0) * inv_sqrt_d + km
    m = jnp.max(sc, axis=1, keepdims=True)
    p = jnp.exp(sc - m)
    p = (p / jnp.sum(p, axis=1, keepdims=True)).astype(bf)
    o_list = []
    for n in range(N):
        vn = jnp.concatenate([v[n * PPAD:(n + 1) * PPAD], zpad], axis=0)
        o_list.append(jnp.dot(p[n * PPAD:(n + 1) * PPAD], vn,
                              preferred_element_type=f32))
    oa = jnp.concatenate(o_list, axis=0).astype(bf)
    o = jnp.dot(oa, wo, preferred_element_type=f32).astype(bf)
    zp2 = jnp.zeros((PLANE - PPAD, D), bf)
    h2_list = []
    for n in range(N):
        on = jnp.concatenate([o[n * PPAD:(n + 1) * PPAD], zp2], axis=0)
        obn = jnp.dot(Re, on, preferred_element_type=f32)
        h2_list.append(xn_list[n] + obn)
    h2 = jnp.concatenate(h2_list, axis=0)
    u = jnp.maximum(jnp.dot(h2.astype(bf), w1_ref[0].astype(bf),
                            preferred_element_type=f32)
                    + b1_ref[0], 0.0)
    y = jnp.dot(u.astype(bf), w2_ref[0].astype(bf),
                preferred_element_type=f32) + b2_ref[0]
    return (h2 + y) * gate


def _expert_kernel(e_ref,
                   xt_ref, g3_ref,
                   wq_a, wk_a, wv_a, wo_a, w1_a, b1_a, w2_a, b2_a,
                   a_a, r_a, km_a,
                   wq_b, wk_b, wv_b, wo_b, w1_b, b1_b, w2_b, b2_b,
                   a_b, r_b, km_b,
                   ewt_ref, endb_ref,
                   out_ref):
    xt3 = jnp.transpose(xt_ref[0], (1, 0, 2))
    xn_list = [xt3[n] for n in range(N)]
    tot_a = _one_expert(xn_list, g3_ref[0, 0, 0],
                        (wq_a, wk_a, wv_a, wo_a, w1_a, b1_a, w2_a, b2_a,
                         a_a, r_a, km_a))
    tot_b = _one_expert(xn_list, g3_ref[0, 0, 1],
                        (wq_b, wk_b, wv_b, wo_b, w1_b, b1_b, w2_b, b2_b,
                         a_b, r_b, km_b))
    tot = tot_a + tot_b
    ewt = ewt_ref[...]
    endb = endb_ref[...]
    rows = []
    for n in range(N):
        resn = jnp.dot(ewt, xn_list[n], preferred_element_type=f32)
        rows.append(resn + endb + tot[n * L:(n + 1) * L])
    outv = jnp.stack(rows, axis=0)
    out_ref[0] = jnp.transpose(outv, (1, 0, 2))


def _gate_call(x0t, wtile, bstart, wgp):
    return pl.pallas_call(
        _gate_kernel,
        out_shape=[
            jax.ShapeDtypeStruct((B, FPAD), i32),
            jax.ShapeDtypeStruct((B, FPAD), f32),
            jax.ShapeDtypeStruct((8, FPAD), f32),
        ],
    )(x0t, _TT, _CT, _ST, _CIT, _SIT, wtile, bstart, wgp)


def _expert_call(e_flat, xt, g3, Wq, Wk, Wv, Wo, W1, b1r, W2, b2r, ewt, endb):
    bspec = lambda shape, imap: pl.BlockSpec(shape, imap)
    by_ea = lambda s, e_ref: (e_ref[2 * s], 0, 0)
    by_eb = lambda s, e_ref: (e_ref[2 * s + 1], 0, 0)

    def wspecs(by_e):
        return [
            bspec((1, D, D), by_e),
            bspec((1, D, D), by_e),
            bspec((1, D, D), by_e),
            bspec((1, D, D), by_e),
            bspec((1, D, DFF), by_e),
            bspec((1, 1, DFF), by_e),
            bspec((1, DFF, D), by_e),
            bspec((1, 1, D), by_e),
            bspec((1, PPAD, L), by_e),
            bspec((1, L, PLANE), by_e),
            bspec((1, 1, PLANE), by_e),
        ]

    grid_spec = pltpu.PrefetchScalarGridSpec(
        num_scalar_prefetch=1,
        grid=(B,),
        in_specs=(
            [bspec((1, L, N, D), lambda s, e_ref: (s, 0, 0, 0)),
             bspec((1, 1, FPAD), lambda s, e_ref: (s, 0, 0))]
            + wspecs(by_ea) + wspecs(by_eb)
            + [bspec((L, L), lambda s, e_ref: (0, 0)),
               bspec((L, D), lambda s, e_ref: (0, 0))]
        ),
        out_specs=pl.BlockSpec((1, L, N, D), lambda s, e_ref: (s, 0, 0, 0)),
    )
    wargs = (Wq, Wk, Wv, Wo, W1, b1r, W2, b2r, _AP, _RP, _KM)
    return pl.pallas_call(
        _expert_kernel,
        grid_spec=grid_spec,
        out_shape=jax.ShapeDtypeStruct((B, L, N, D), f32),
    )(e_flat, xt, g3, *wargs, *wargs, ewt, endb)


def kernel(x, W_start, b_start, w_gate, Wq, Wk, Wv, Wo, W1, b1, W2, b2, end_W, end_b):
    x0t = x[:, :, :, 0].transpose(0, 2, 1).reshape(B * N, L)
    wtile = jnp.broadcast_to(jnp.tile(W_start[:, 0], B)[None, :], (8, B * N))
    bstart = jnp.broadcast_to(b_start.reshape(1, 1), (8, FPAD))
    wgp = jnp.pad(w_gate, ((0, 0), (0, FPAD - E)))
    idx, gate, bal = _gate_call(x0t, wtile, bstart, wgp)

    e_flat = idx[:, :K].reshape(-1)
    g3 = gate[:, None, :]
    b1r = b1[:, None, :]
    b2r = b2[:, None, :]
    ewt = end_W.T
    endb = jnp.broadcast_to(end_b[:, None], (L, D))
    out = _expert_call(e_flat, x, g3, Wq, Wk, Wv, Wo, W1, b1r, W2, b2r, ewt, endb)
    return out, bal[0, 0]
